# baseline (speedup 1.0000x reference)
import math

import jax
import jax.numpy as jnp
from jax import lax
from jax.experimental import pallas as pl
from jax.experimental.pallas import tpu as pltpu
from jax.experimental.pallas import tpu_sc as plsc

NUM_EMBEDDINGS = 1000000
D_MODEL = 64
SCALE = math.sqrt(D_MODEL)

B_ROWS = 4096
B_COLS = 50
N_TOTAL = B_ROWS * B_COLS

_INFO = plsc.get_sparse_core_info()
NC = _INFO.num_cores
NS = _INFO.num_subcores
NW = NC * NS
LANES = _INFO.num_lanes

SUB = 64
K_PER_CHUNK = 10
CHUNK = SUB * K_PER_CHUNK
PER_W = N_TOTAL // NW
NCHUNK = PER_W // CHUNK
ROWS_PER_W = PER_W // SUB


def _body(table_hbm, idx_hbm, out_hbm, idx_v, buf0, buf1,
          sem_g0, sem_g1, sem_w0, sem_w1):
    wid = lax.axis_index("s") * NC + lax.axis_index("c")
    out_base = wid * PER_W

    pltpu.sync_copy(idx_hbm.at[wid], idx_v)

    bufs = (buf0, buf1)
    sems_g = (sem_g0, sem_g1)
    sems_w = (sem_w0, sem_w1)

    def fire(ci, b):
        for j in range(K_PER_CHUNK):
            pltpu.async_copy(table_hbm.at[idx_v.at[ci * K_PER_CHUNK + j]],
                             bufs[b].at[pl.ds(j * SUB, SUB)], sems_g[b])

    def drain(ci, b):
        for j in range(K_PER_CHUNK):
            pltpu.make_async_copy(table_hbm.at[idx_v.at[ci * K_PER_CHUNK + j]],
                                  bufs[b].at[pl.ds(j * SUB, SUB)],
                                  sems_g[b]).wait()

    def scale(b):
        buf = bufs[b]

        @plsc.parallel_loop(0, CHUNK, unroll=8)
        def _r(r):
            for v in range(D_MODEL // LANES):
                sl = pl.ds(v * LANES, LANES)
                buf[r, sl] = buf[r, sl] * SCALE

    def out_start(ci, b):
        pltpu.async_copy(bufs[b],
                         out_hbm.at[pl.ds(out_base + ci * CHUNK, CHUNK)],
                         sems_w[b])

    def out_wait(ci, b):
        pltpu.make_async_copy(bufs[b],
                              out_hbm.at[pl.ds(out_base + ci * CHUNK, CHUNK)],
                              sems_w[b]).wait()

    fire(0, 0)
    fire(1, 1)

    def pair(p, _):
        for b in range(2):
            ci = 2 * p + b
            drain(ci, b)
            scale(b)
            out_start(ci, b)
        for b in range(2):
            ci = 2 * p + b
            out_wait(ci, b)
            fire(ci + 2, b)
        return 0

    lax.fori_loop(0, NCHUNK // 2 - 1, pair, 0)

    for b in range(2):
        ci = NCHUNK - 2 + b
        drain(ci, b)
        scale(b)
        out_start(ci, b)
    for b in range(2):
        out_wait(NCHUNK - 2 + b, b)


@jax.jit
def _embed(table, idx3d):
    mesh = plsc.VectorSubcoreMesh(core_axis_name="c", subcore_axis_name="s")
    kern = pl.kernel(
        _body,
        out_type=jax.ShapeDtypeStruct((N_TOTAL, D_MODEL), jnp.float32),
        mesh=mesh,
        scratch_types=[
            pltpu.VMEM((ROWS_PER_W, SUB), jnp.int32),
            pltpu.VMEM((CHUNK, D_MODEL), jnp.float32),
            pltpu.VMEM((CHUNK, D_MODEL), jnp.float32),
            pltpu.SemaphoreType.DMA,
            pltpu.SemaphoreType.DMA,
            pltpu.SemaphoreType.DMA,
            pltpu.SemaphoreType.DMA,
        ],
        compiler_params=pltpu.CompilerParams(use_tc_tiling_on_sc=False),
    )
    return kern(table, idx3d)


def kernel(inputs, table):
    idx3d = inputs.reshape(NW, ROWS_PER_W, SUB).astype(jnp.int32)
    out = _embed(table, idx3d)
    return out.reshape(B_ROWS, B_COLS, D_MODEL)
